# Initial kernel scaffold; baseline (speedup 1.0000x reference)
import jax
import jax.numpy as jnp
from jax.experimental import pallas as pl


def kernel(x, W):
    raise NotImplementedError("write your pallas kernel here")


# speedup vs baseline: 1.3583x; 1.3583x over previous
import functools

import jax
import jax.numpy as jnp
from jax import lax
from jax.experimental import pallas as pl
from jax.experimental.pallas import tpu as pltpu


def _gate_kernel(x_ref, wt_ref, tp_ref, ti_ref, aux_ref, mask_ref,
                 imp_ref, load_ref, *, n_tokens):
    i = pl.program_id(0)
    nsteps = pl.num_programs(0)

    logits = jnp.dot(x_ref[...], wt_ref[...],
                     preferred_element_type=jnp.float32)
    m = jnp.max(logits, axis=-1, keepdims=True)
    e = jnp.exp(logits - m)
    s = jnp.sum(e, axis=-1, keepdims=True)
    probs = e / s

    bt, ne = logits.shape
    col = lax.broadcasted_iota(jnp.int32, (bt, ne), 1)

    p1 = jnp.max(probs, axis=-1, keepdims=True)
    i1 = jnp.min(jnp.where(probs == p1, col, ne), axis=-1, keepdims=True)
    hit1 = col == i1
    probs_m = jnp.where(hit1, -1.0, probs)
    p2 = jnp.max(probs_m, axis=-1, keepdims=True)
    i2 = jnp.min(jnp.where(probs_m == p2, col, ne), axis=-1, keepdims=True)
    hit2 = col == i2

    denom = p1 + p2
    tp_ref[:, 0:1] = p1 / denom
    tp_ref[:, 1:2] = p2 / denom
    ti_ref[:, 0:1] = i1
    ti_ref[:, 1:2] = i2

    mask = (hit1 | hit2).astype(jnp.float32)
    mask_ref[...] = mask

    imp_part = jnp.sum(probs, axis=0, keepdims=True)
    load_part = jnp.sum(mask, axis=0, keepdims=True)

    @pl.when(i == 0)
    def _():
        imp_ref[...] = imp_part
        load_ref[...] = load_part
        aux_ref[...] = jnp.zeros_like(aux_ref)

    @pl.when(i > 0)
    def _():
        imp_ref[...] += imp_part
        load_ref[...] += load_part

    @pl.when(i == nsteps - 1)
    def _():
        scale = ne / (n_tokens * n_tokens + 1e-06)
        aux_ref[...] = jnp.sum(imp_ref[...] * load_ref[...],
                               keepdims=True).reshape(1, 1) * scale


def kernel(x, W):
    n, d = x.shape
    ne = W.shape[0]
    bt = 512
    nsteps = n // bt
    wt = W.T

    top_probs, top_indices, aux, mask = pl.pallas_call(
        functools.partial(_gate_kernel, n_tokens=n),
        grid=(nsteps,),
        in_specs=[
            pl.BlockSpec((bt, d), lambda i: (i, 0)),
            pl.BlockSpec((d, ne), lambda i: (0, 0)),
        ],
        out_specs=[
            pl.BlockSpec((bt, 2), lambda i: (i, 0)),
            pl.BlockSpec((bt, 2), lambda i: (i, 0)),
            pl.BlockSpec((1, 1), lambda i: (0, 0)),
            pl.BlockSpec((bt, ne), lambda i: (i, 0)),
        ],
        out_shape=[
            jax.ShapeDtypeStruct((n, 2), jnp.float32),
            jax.ShapeDtypeStruct((n, 2), jnp.int32),
            jax.ShapeDtypeStruct((1, 1), jnp.float32),
            jax.ShapeDtypeStruct((n, ne), jnp.float32),
        ],
        scratch_shapes=[
            pltpu.VMEM((1, ne), jnp.float32),
            pltpu.VMEM((1, ne), jnp.float32),
        ],
    )(x, wt)

    return top_probs, top_indices, aux[0, 0], mask


# speedup vs baseline: 1.4557x; 1.0717x over previous
import functools

import jax
import jax.numpy as jnp
from jax import lax
from jax.experimental import pallas as pl
from jax.experimental.pallas import tpu as pltpu


def _gate_kernel(x_ref, wt_ref, tp_ref, ti_ref, aux_ref, mask_ref,
                 imp_ref, load_ref, *, n_tokens):
    i = pl.program_id(0)
    nsteps = pl.num_programs(0)

    logits = jnp.dot(x_ref[...], wt_ref[...],
                     preferred_element_type=jnp.float32)
    m = jnp.max(logits, axis=-1, keepdims=True)
    e = jnp.exp(logits - m)
    s = jnp.sum(e, axis=-1, keepdims=True)
    probs = e / s

    bt, ne = logits.shape
    col = lax.broadcasted_iota(jnp.int32, (bt, ne), 1)

    p1 = jnp.max(probs, axis=-1, keepdims=True)
    i1 = jnp.min(jnp.where(probs == p1, col, ne), axis=-1, keepdims=True)
    hit1 = col == i1
    probs_m = jnp.where(hit1, -1.0, probs)
    p2 = jnp.max(probs_m, axis=-1, keepdims=True)
    i2 = jnp.min(jnp.where(probs_m == p2, col, ne), axis=-1, keepdims=True)
    hit2 = col == i2

    denom = p1 + p2
    tp_ref[:, 0:1] = p1 / denom
    tp_ref[:, 1:2] = p2 / denom
    ti_ref[:, 0:1] = i1
    ti_ref[:, 1:2] = i2

    mask = (hit1 | hit2).astype(jnp.float32)
    mask_ref[...] = mask

    imp_part = jnp.sum(probs, axis=0, keepdims=True)
    load_part = jnp.sum(mask, axis=0, keepdims=True)

    @pl.when(i == 0)
    def _():
        imp_ref[...] = imp_part
        load_ref[...] = load_part
        aux_ref[...] = jnp.zeros_like(aux_ref)

    @pl.when(i > 0)
    def _():
        imp_ref[...] += imp_part
        load_ref[...] += load_part

    @pl.when(i == nsteps - 1)
    def _():
        scale = ne / (n_tokens * n_tokens + 1e-06)
        aux_ref[...] = jnp.sum(imp_ref[...] * load_ref[...],
                               keepdims=True).reshape(1, 1) * scale


def kernel(x, W):
    n, d = x.shape
    ne = W.shape[0]
    bt = 1024
    nsteps = n // bt
    wt = W.T

    top_probs, top_indices, aux, mask = pl.pallas_call(
        functools.partial(_gate_kernel, n_tokens=n),
        grid=(nsteps,),
        in_specs=[
            pl.BlockSpec((bt, d), lambda i: (i, 0)),
            pl.BlockSpec((d, ne), lambda i: (0, 0)),
        ],
        out_specs=[
            pl.BlockSpec((bt, 2), lambda i: (i, 0)),
            pl.BlockSpec((bt, 2), lambda i: (i, 0)),
            pl.BlockSpec((1, 1), lambda i: (0, 0)),
            pl.BlockSpec((bt, ne), lambda i: (i, 0)),
        ],
        out_shape=[
            jax.ShapeDtypeStruct((n, 2), jnp.float32),
            jax.ShapeDtypeStruct((n, 2), jnp.int32),
            jax.ShapeDtypeStruct((1, 1), jnp.float32),
            jax.ShapeDtypeStruct((n, ne), jnp.float32),
        ],
        scratch_shapes=[
            pltpu.VMEM((1, ne), jnp.float32),
            pltpu.VMEM((1, ne), jnp.float32),
        ],
    )(x, wt)

    return top_probs, top_indices, aux[0, 0], mask
